# baseline (speedup 1.0000x reference)
import jax
import jax.numpy as jnp
from jax.experimental import pallas as pl
from jax.experimental.pallas import tpu as pltpu

NBLK = 8192


def _fused_body(spikes_ref, rates_ref, out_ref, cnt_ref):
    i = pl.program_id(0)

    r = jnp.transpose(rates_ref[...])
    k = r.shape[0]

    norm = jnp.sum(jnp.abs(r), axis=0, keepdims=True)
    mx = jnp.max(r, axis=0, keepdims=True)
    sub = jax.lax.broadcasted_iota(jnp.int32, r.shape, 0)
    idx = jnp.min(jnp.where(r == mx, sub, k), axis=0, keepdims=True)
    w = mx / jnp.maximum(norm, 1e-12)

    hit = sub == idx
    oh = jnp.where(hit, w, 0.0).astype(jnp.bfloat16)
    ohc = jnp.where(hit, 1.0, 0.0).astype(jnp.bfloat16)

    part = jax.lax.dot_general(
        spikes_ref[...].astype(jnp.bfloat16), oh, (((1,), (1,)), ((), ())),
        preferred_element_type=jnp.float32)
    ones = jnp.ones((8, NBLK), jnp.bfloat16)
    cpart = jax.lax.dot_general(
        ones, ohc, (((1,), (1,)), ((), ())),
        preferred_element_type=jnp.float32)

    @pl.when(i == 0)
    def _():
        out_ref[...] = jnp.zeros_like(out_ref)
        cnt_ref[...] = jnp.zeros_like(cnt_ref)

    out_ref[...] += part
    cnt_ref[...] += cpart

    @pl.when(i == pl.num_programs(0) - 1)
    def _():
        cnt = cnt_ref[0:1, :]
        acc = out_ref[...]
        out_ref[...] = jnp.where(cnt > 0.0, acc / cnt, 0.0)


def kernel(spikes, rates):
    b, n = spikes.shape
    k = rates.shape[1]

    return pl.pallas_call(
        _fused_body,
        grid=(n // NBLK,),
        in_specs=[
            pl.BlockSpec((b, NBLK), lambda i: (0, i)),
            pl.BlockSpec((NBLK, k), lambda i: (i, 0)),
        ],
        out_specs=pl.BlockSpec((b, k), lambda i: (0, 0)),
        out_shape=jax.ShapeDtypeStruct((b, k), jnp.float32),
        scratch_shapes=[pltpu.VMEM((8, k), jnp.float32)],
        compiler_params=pltpu.CompilerParams(
            dimension_semantics=("arbitrary",),
        ),
    )(spikes, rates)


# speedup vs baseline: 2.0644x; 2.0644x over previous
import jax
import jax.numpy as jnp
from jax.experimental import pallas as pl
from jax.experimental.pallas import tpu as pltpu

NBLK = 8192
KP = 16


def _fused_body(s_top_ref, s_bot_ref, ratesT_ref, out_ref, cnt_ref):
    i = pl.program_id(0)

    r = ratesT_ref[...]
    kk = r.shape[0]

    norm = jnp.sum(jnp.abs(r), axis=0, keepdims=True)
    mx = jnp.max(r, axis=0, keepdims=True)
    sub = jax.lax.broadcasted_iota(jnp.int32, r.shape, 0)
    idx = jnp.min(jnp.where(r == mx, sub, kk), axis=0, keepdims=True)
    w = mx / jnp.maximum(norm, 1e-12)

    hit = sub == idx
    oh = jnp.where(hit, w, 0.0).astype(jnp.bfloat16)
    ohc = jnp.where(hit, 1.0, 0.0).astype(jnp.bfloat16)

    hb = out_ref.shape[0] // 2
    top = jax.lax.dot_general(
        s_top_ref[...].astype(jnp.bfloat16), oh, (((1,), (1,)), ((), ())),
        preferred_element_type=jnp.float32)
    bot = jax.lax.dot_general(
        s_bot_ref[...].astype(jnp.bfloat16), oh, (((1,), (1,)), ((), ())),
        preferred_element_type=jnp.float32)
    ones = jnp.ones((8, NBLK), jnp.bfloat16)
    cpart = jax.lax.dot_general(
        ones, ohc, (((1,), (1,)), ((), ())),
        preferred_element_type=jnp.float32)

    @pl.when(i == 0)
    def _():
        out_ref[...] = jnp.zeros_like(out_ref)
        cnt_ref[...] = jnp.zeros_like(cnt_ref)

    out_ref[0:hb, :] += top
    out_ref[hb:, :] += bot
    cnt_ref[:, 0:cpart.shape[1]] += cpart

    @pl.when(i == pl.num_programs(0) - 1)
    def _():
        cnt = cnt_ref[0:1, 0:out_ref.shape[1]]
        acc = out_ref[...]
        out_ref[...] = jnp.where(cnt > 0.0, acc / cnt, 0.0)


def kernel(spikes, rates):
    b, n = spikes.shape
    k = rates.shape[1]
    hb = b // 2

    ratesT = rates.T

    out = pl.pallas_call(
        _fused_body,
        grid=(n // NBLK,),
        in_specs=[
            pl.BlockSpec((hb, NBLK), lambda i: (0, i)),
            pl.BlockSpec((hb, NBLK), lambda i: (1, i)),
            pl.BlockSpec((k, NBLK), lambda i: (0, i)),
        ],
        out_specs=pl.BlockSpec((b, k), lambda i: (0, 0)),
        out_shape=jax.ShapeDtypeStruct((b, k), jnp.float32),
        scratch_shapes=[pltpu.VMEM((8, 16), jnp.float32)],
        compiler_params=pltpu.CompilerParams(
            dimension_semantics=("arbitrary",),
        ),
    )(spikes, spikes, ratesT)
    return out
